# initial kernel scaffold (unmeasured)
import jax
import jax.numpy as jnp
from jax import lax
from jax.experimental import pallas as pl
from jax.experimental.pallas import tpu as pltpu


def kernel(
    u,
):
    def body(*refs):
        pass

    out_shape = jax.ShapeDtypeStruct(..., jnp.float32)
    return pl.pallas_call(body, out_shape=out_shape)(...)


# baseline (device time: 12651 ns/iter reference)
import jax
import jax.numpy as jnp
from jax import lax
from jax.experimental import pallas as pl
from jax.experimental.pallas import tpu as pltpu

NX, NY, NZ = 2, 2, 4
S = 64


def kernel(u):
    s = S
    zlo = u[:, :, 0]
    zhi = u[:, :, s - 1]

    def body(u_ref, zlo_ref, zhi_ref, o_ref, rzl_ref, rzh_ref,
             bxl, bxh, byl, byh, send_sems, recv_sems):
        mx = lax.axis_index("x")
        my = lax.axis_index("y")
        mz = lax.axis_index("z")

        has_xm = mx > 0
        has_xp = mx < NX - 1
        has_ym = my > 0
        has_yp = my < NY - 1
        has_zm = mz > 0
        has_zp = mz < NZ - 1

        barrier_sem = pltpu.get_barrier_semaphore()

        def sig(dev):
            pl.semaphore_signal(
                barrier_sem, inc=1, device_id=dev,
                device_id_type=pl.DeviceIdType.MESH,
            )

        @pl.when(has_xm)
        def _():
            sig((mx - 1, my, mz))

        @pl.when(has_xp)
        def _():
            sig((mx + 1, my, mz))

        @pl.when(has_ym)
        def _():
            sig((mx, my - 1, mz))

        @pl.when(has_yp)
        def _():
            sig((mx, my + 1, mz))

        @pl.when(has_zm)
        def _():
            sig((mx, my, mz - 1))

        @pl.when(has_zp)
        def _():
            sig((mx, my, mz + 1))

        pl.semaphore_wait(barrier_sem, 3)

        @pl.when(has_zm & has_zp)
        def _():
            pl.semaphore_wait(barrier_sem, 1)

        def mk(src, dst, ssem_i, rsem_i, dev):
            return pltpu.make_async_remote_copy(
                src_ref=src,
                dst_ref=dst,
                send_sem=send_sems.at[ssem_i],
                recv_sem=recv_sems.at[rsem_i],
                device_id=dev,
                device_id_type=pl.DeviceIdType.MESH,
            )

        @pl.when(has_xm)
        def _():
            mk(u_ref.at[pl.ds(0, 1), :, :], bxh.at[pl.ds(0, 1), :, :],
               0, 1, (mx - 1, my, mz)).start()

        @pl.when(has_xp)
        def _():
            mk(u_ref.at[pl.ds(s - 1, 1), :, :], bxl.at[pl.ds(s - 1, 1), :, :],
               1, 0, (mx + 1, my, mz)).start()

        @pl.when(has_ym)
        def _():
            mk(u_ref.at[:, pl.ds(0, 1), :], byh.at[:, pl.ds(0, 1), :],
               2, 3, (mx, my - 1, mz)).start()

        @pl.when(has_yp)
        def _():
            mk(u_ref.at[:, pl.ds(s - 1, 1), :], byl.at[:, pl.ds(s - 1, 1), :],
               3, 2, (mx, my + 1, mz)).start()

        @pl.when(has_zm)
        def _():
            mk(zlo_ref, rzh_ref, 4, 5, (mx, my, mz - 1)).start()

        @pl.when(has_zp)
        def _():
            mk(zhi_ref, rzl_ref, 5, 4, (mx, my, mz + 1)).start()

        o_ref[...] = -6.0 * u_ref[...]
        o_ref[pl.ds(1, s - 1), :, :] = (
            o_ref[pl.ds(1, s - 1), :, :] + u_ref[pl.ds(0, s - 1), :, :]
        )
        o_ref[pl.ds(0, s - 1), :, :] = (
            o_ref[pl.ds(0, s - 1), :, :] + u_ref[pl.ds(1, s - 1), :, :]
        )
        o_ref[:, pl.ds(1, s - 1), :] = (
            o_ref[:, pl.ds(1, s - 1), :] + u_ref[:, pl.ds(0, s - 1), :]
        )
        o_ref[:, pl.ds(0, s - 1), :] = (
            o_ref[:, pl.ds(0, s - 1), :] + u_ref[:, pl.ds(1, s - 1), :]
        )
        o_ref[:, :, pl.ds(1, s - 1)] = (
            o_ref[:, :, pl.ds(1, s - 1)] + u_ref[:, :, pl.ds(0, s - 1)]
        )
        o_ref[:, :, pl.ds(0, s - 1)] = (
            o_ref[:, :, pl.ds(0, s - 1)] + u_ref[:, :, pl.ds(1, s - 1)]
        )

        def wait_recv(dst, rsem_i):
            pltpu.make_async_remote_copy(
                src_ref=dst,
                dst_ref=dst,
                send_sem=send_sems.at[rsem_i],
                recv_sem=recv_sems.at[rsem_i],
                device_id=(mx, my, mz),
                device_id_type=pl.DeviceIdType.MESH,
            ).wait_recv()

        @pl.when(has_xm)
        def _():
            wait_recv(bxl.at[pl.ds(s - 1, 1), :, :], 0)
            o_ref[pl.ds(0, 1), :, :] = (
                o_ref[pl.ds(0, 1), :, :] + bxl[pl.ds(s - 1, 1), :, :]
            )

        @pl.when(has_xp)
        def _():
            wait_recv(bxh.at[pl.ds(0, 1), :, :], 1)
            o_ref[pl.ds(s - 1, 1), :, :] = (
                o_ref[pl.ds(s - 1, 1), :, :] + bxh[pl.ds(0, 1), :, :]
            )

        @pl.when(has_ym)
        def _():
            wait_recv(byl.at[:, pl.ds(s - 1, 1), :], 2)
            o_ref[:, pl.ds(0, 1), :] = (
                o_ref[:, pl.ds(0, 1), :] + byl[:, pl.ds(s - 1, 1), :]
            )

        @pl.when(has_yp)
        def _():
            wait_recv(byh.at[:, pl.ds(0, 1), :], 3)
            o_ref[:, pl.ds(s - 1, 1), :] = (
                o_ref[:, pl.ds(s - 1, 1), :] + byh[:, pl.ds(0, 1), :]
            )

        @pl.when(has_zm)
        def _():
            wait_recv(rzl_ref, 4)

        @pl.when(has_zp)
        def _():
            wait_recv(rzh_ref, 5)

        @pl.when(mx == 0)
        def _():
            o_ref[pl.ds(0, 1), :, :] = jnp.zeros((1, s, s), jnp.float32)

        @pl.when(mx == NX - 1)
        def _():
            o_ref[pl.ds(s - 1, 1), :, :] = jnp.zeros((1, s, s), jnp.float32)

        @pl.when(my == 0)
        def _():
            o_ref[:, pl.ds(0, 1), :] = jnp.zeros((s, 1, s), jnp.float32)

        @pl.when(my == NY - 1)
        def _():
            o_ref[:, pl.ds(s - 1, 1), :] = jnp.zeros((s, 1, s), jnp.float32)

        @pl.when(mz == 0)
        def _():
            o_ref[:, :, pl.ds(0, 1)] = jnp.zeros((s, s, 1), jnp.float32)

        @pl.when(mz == NZ - 1)
        def _():
            o_ref[:, :, pl.ds(s - 1, 1)] = jnp.zeros((s, s, 1), jnp.float32)

        def wait_send(src, ssem_i):
            pltpu.make_async_remote_copy(
                src_ref=src,
                dst_ref=src,
                send_sem=send_sems.at[ssem_i],
                recv_sem=recv_sems.at[ssem_i],
                device_id=(mx, my, mz),
                device_id_type=pl.DeviceIdType.MESH,
            ).wait_send()

        @pl.when(has_xm)
        def _():
            wait_send(u_ref.at[pl.ds(0, 1), :, :], 0)

        @pl.when(has_xp)
        def _():
            wait_send(u_ref.at[pl.ds(s - 1, 1), :, :], 1)

        @pl.when(has_ym)
        def _():
            wait_send(u_ref.at[:, pl.ds(0, 1), :], 2)

        @pl.when(has_yp)
        def _():
            wait_send(u_ref.at[:, pl.ds(s - 1, 1), :], 3)

        @pl.when(has_zm)
        def _():
            wait_send(zlo_ref, 4)

        @pl.when(has_zp)
        def _():
            wait_send(zhi_ref, 5)

    out, rzl, rzh = pl.pallas_call(
        body,
        out_shape=[
            jax.ShapeDtypeStruct((s, s, s), jnp.float32),
            jax.ShapeDtypeStruct((s, s), jnp.float32),
            jax.ShapeDtypeStruct((s, s), jnp.float32),
        ],
        in_specs=[
            pl.BlockSpec(memory_space=pltpu.VMEM),
            pl.BlockSpec(memory_space=pltpu.VMEM),
            pl.BlockSpec(memory_space=pltpu.VMEM),
        ],
        out_specs=[
            pl.BlockSpec(memory_space=pltpu.VMEM),
            pl.BlockSpec(memory_space=pltpu.VMEM),
            pl.BlockSpec(memory_space=pltpu.VMEM),
        ],
        scratch_shapes=[
            pltpu.VMEM((s, s, s), jnp.float32),
            pltpu.VMEM((s, s, s), jnp.float32),
            pltpu.VMEM((s, s, s), jnp.float32),
            pltpu.VMEM((s, s, s), jnp.float32),
            pltpu.SemaphoreType.DMA((6,)),
            pltpu.SemaphoreType.DMA((6,)),
        ],
        compiler_params=pltpu.CompilerParams(collective_id=0),
    )(u, zlo, zhi)

    mx = lax.axis_index("x")
    my = lax.axis_index("y")
    mz = lax.axis_index("z")
    kidx = jnp.arange(s)
    ii = jnp.arange(s)[:, None]
    jj = jnp.arange(s)[None, :]
    ok_i = ((mx > 0) | (ii > 0)) & ((mx < NX - 1) | (ii < s - 1))
    ok_j = ((my > 0) | (jj > 0)) & ((my < NY - 1) | (jj < s - 1))
    mask2d = ok_i & ok_j
    add_lo = jnp.where((mz > 0) & mask2d, rzl, 0.0)[:, :, None] * (kidx == 0)
    add_hi = jnp.where((mz < NZ - 1) & mask2d, rzh, 0.0)[:, :, None] * (kidx == s - 1)
    return out + add_lo + add_hi


# device time: 10164 ns/iter; 1.2447x vs baseline; 1.2447x over previous
import jax
import jax.numpy as jnp
from jax import lax
from jax.experimental import pallas as pl
from jax.experimental.pallas import tpu as pltpu

NX, NY, NZ = 2, 2, 4
S = 64


def kernel(u):
    s = S

    def body(u_ref, o_ref, zlo_ref, zhi_ref, rzl_ref, rzh_ref,
             bxl, bxh, byl, byh, send_sems, recv_sems):
        mx = lax.axis_index("x")
        my = lax.axis_index("y")
        mz = lax.axis_index("z")

        has_xm = mx > 0
        has_xp = mx < NX - 1
        has_ym = my > 0
        has_yp = my < NY - 1
        has_zm = mz > 0
        has_zp = mz < NZ - 1

        zlo_ref[...] = u_ref[:, :, 0]
        zhi_ref[...] = u_ref[:, :, s - 1]

        barrier_sem = pltpu.get_barrier_semaphore()

        def sig(dev):
            pl.semaphore_signal(
                barrier_sem, inc=1, device_id=dev,
                device_id_type=pl.DeviceIdType.MESH,
            )

        @pl.when(has_xm)
        def _():
            sig((mx - 1, my, mz))

        @pl.when(has_xp)
        def _():
            sig((mx + 1, my, mz))

        @pl.when(has_ym)
        def _():
            sig((mx, my - 1, mz))

        @pl.when(has_yp)
        def _():
            sig((mx, my + 1, mz))

        @pl.when(has_zm)
        def _():
            sig((mx, my, mz - 1))

        @pl.when(has_zp)
        def _():
            sig((mx, my, mz + 1))

        pl.semaphore_wait(barrier_sem, 3)

        @pl.when(has_zm & has_zp)
        def _():
            pl.semaphore_wait(barrier_sem, 1)

        def mk(src, dst, ssem_i, rsem_i, dev):
            return pltpu.make_async_remote_copy(
                src_ref=src,
                dst_ref=dst,
                send_sem=send_sems.at[ssem_i],
                recv_sem=recv_sems.at[rsem_i],
                device_id=dev,
                device_id_type=pl.DeviceIdType.MESH,
            )

        @pl.when(has_xm)
        def _():
            mk(u_ref.at[pl.ds(0, 1), :, :], bxh.at[pl.ds(0, 1), :, :],
               0, 1, (mx - 1, my, mz)).start()

        @pl.when(has_xp)
        def _():
            mk(u_ref.at[pl.ds(s - 1, 1), :, :], bxl.at[pl.ds(s - 1, 1), :, :],
               1, 0, (mx + 1, my, mz)).start()

        @pl.when(has_ym)
        def _():
            mk(u_ref.at[:, pl.ds(0, 1), :], byh.at[:, pl.ds(0, 1), :],
               2, 3, (mx, my - 1, mz)).start()

        @pl.when(has_yp)
        def _():
            mk(u_ref.at[:, pl.ds(s - 1, 1), :], byl.at[:, pl.ds(s - 1, 1), :],
               3, 2, (mx, my + 1, mz)).start()

        @pl.when(has_zm)
        def _():
            mk(zlo_ref, rzh_ref, 4, 5, (mx, my, mz - 1)).start()

        @pl.when(has_zp)
        def _():
            mk(zhi_ref, rzl_ref, 5, 4, (mx, my, mz + 1)).start()

        o_ref[...] = -6.0 * u_ref[...]
        o_ref[pl.ds(1, s - 1), :, :] = (
            o_ref[pl.ds(1, s - 1), :, :] + u_ref[pl.ds(0, s - 1), :, :]
        )
        o_ref[pl.ds(0, s - 1), :, :] = (
            o_ref[pl.ds(0, s - 1), :, :] + u_ref[pl.ds(1, s - 1), :, :]
        )
        o_ref[:, pl.ds(1, s - 1), :] = (
            o_ref[:, pl.ds(1, s - 1), :] + u_ref[:, pl.ds(0, s - 1), :]
        )
        o_ref[:, pl.ds(0, s - 1), :] = (
            o_ref[:, pl.ds(0, s - 1), :] + u_ref[:, pl.ds(1, s - 1), :]
        )
        o_ref[:, :, pl.ds(1, s - 1)] = (
            o_ref[:, :, pl.ds(1, s - 1)] + u_ref[:, :, pl.ds(0, s - 1)]
        )
        o_ref[:, :, pl.ds(0, s - 1)] = (
            o_ref[:, :, pl.ds(0, s - 1)] + u_ref[:, :, pl.ds(1, s - 1)]
        )

        def wait_recv(dst, rsem_i):
            pltpu.make_async_remote_copy(
                src_ref=dst,
                dst_ref=dst,
                send_sem=send_sems.at[rsem_i],
                recv_sem=recv_sems.at[rsem_i],
                device_id=(mx, my, mz),
                device_id_type=pl.DeviceIdType.MESH,
            ).wait_recv()

        @pl.when(has_xm)
        def _():
            wait_recv(bxl.at[pl.ds(s - 1, 1), :, :], 0)
            o_ref[pl.ds(0, 1), :, :] = (
                o_ref[pl.ds(0, 1), :, :] + bxl[pl.ds(s - 1, 1), :, :]
            )

        @pl.when(has_xp)
        def _():
            wait_recv(bxh.at[pl.ds(0, 1), :, :], 1)
            o_ref[pl.ds(s - 1, 1), :, :] = (
                o_ref[pl.ds(s - 1, 1), :, :] + bxh[pl.ds(0, 1), :, :]
            )

        @pl.when(has_ym)
        def _():
            wait_recv(byl.at[:, pl.ds(s - 1, 1), :], 2)
            o_ref[:, pl.ds(0, 1), :] = (
                o_ref[:, pl.ds(0, 1), :] + byl[:, pl.ds(s - 1, 1), :]
            )

        @pl.when(has_yp)
        def _():
            wait_recv(byh.at[:, pl.ds(0, 1), :], 3)
            o_ref[:, pl.ds(s - 1, 1), :] = (
                o_ref[:, pl.ds(s - 1, 1), :] + byh[:, pl.ds(0, 1), :]
            )

        @pl.when(has_zm)
        def _():
            wait_recv(rzl_ref, 4)
            o_ref[:, :, pl.ds(0, 1)] = (
                o_ref[:, :, pl.ds(0, 1)] + rzl_ref[...][:, :, None]
            )

        @pl.when(has_zp)
        def _():
            wait_recv(rzh_ref, 5)
            o_ref[:, :, pl.ds(s - 1, 1)] = (
                o_ref[:, :, pl.ds(s - 1, 1)] + rzh_ref[...][:, :, None]
            )

        @pl.when(mx == 0)
        def _():
            o_ref[pl.ds(0, 1), :, :] = jnp.zeros((1, s, s), jnp.float32)

        @pl.when(mx == NX - 1)
        def _():
            o_ref[pl.ds(s - 1, 1), :, :] = jnp.zeros((1, s, s), jnp.float32)

        @pl.when(my == 0)
        def _():
            o_ref[:, pl.ds(0, 1), :] = jnp.zeros((s, 1, s), jnp.float32)

        @pl.when(my == NY - 1)
        def _():
            o_ref[:, pl.ds(s - 1, 1), :] = jnp.zeros((s, 1, s), jnp.float32)

        @pl.when(mz == 0)
        def _():
            o_ref[:, :, pl.ds(0, 1)] = jnp.zeros((s, s, 1), jnp.float32)

        @pl.when(mz == NZ - 1)
        def _():
            o_ref[:, :, pl.ds(s - 1, 1)] = jnp.zeros((s, s, 1), jnp.float32)

        def wait_send(src, ssem_i):
            pltpu.make_async_remote_copy(
                src_ref=src,
                dst_ref=src,
                send_sem=send_sems.at[ssem_i],
                recv_sem=recv_sems.at[ssem_i],
                device_id=(mx, my, mz),
                device_id_type=pl.DeviceIdType.MESH,
            ).wait_send()

        @pl.when(has_xm)
        def _():
            wait_send(u_ref.at[pl.ds(0, 1), :, :], 0)

        @pl.when(has_xp)
        def _():
            wait_send(u_ref.at[pl.ds(s - 1, 1), :, :], 1)

        @pl.when(has_ym)
        def _():
            wait_send(u_ref.at[:, pl.ds(0, 1), :], 2)

        @pl.when(has_yp)
        def _():
            wait_send(u_ref.at[:, pl.ds(s - 1, 1), :], 3)

        @pl.when(has_zm)
        def _():
            wait_send(zlo_ref, 4)

        @pl.when(has_zp)
        def _():
            wait_send(zhi_ref, 5)

    return pl.pallas_call(
        body,
        out_shape=jax.ShapeDtypeStruct((s, s, s), jnp.float32),
        in_specs=[pl.BlockSpec(memory_space=pltpu.VMEM)],
        out_specs=pl.BlockSpec(memory_space=pltpu.VMEM),
        scratch_shapes=[
            pltpu.VMEM((s, s), jnp.float32),
            pltpu.VMEM((s, s), jnp.float32),
            pltpu.VMEM((s, s), jnp.float32),
            pltpu.VMEM((s, s), jnp.float32),
            pltpu.VMEM((s, s, s), jnp.float32),
            pltpu.VMEM((s, s, s), jnp.float32),
            pltpu.VMEM((s, s, s), jnp.float32),
            pltpu.VMEM((s, s, s), jnp.float32),
            pltpu.SemaphoreType.DMA((6,)),
            pltpu.SemaphoreType.DMA((6,)),
        ],
        compiler_params=pltpu.CompilerParams(collective_id=0),
    )(u)


# device time: 8953 ns/iter; 1.4130x vs baseline; 1.1353x over previous
import jax
import jax.numpy as jnp
from jax import lax
from jax.experimental import pallas as pl
from jax.experimental.pallas import tpu as pltpu

NX, NY, NZ = 2, 2, 4
S = 64


def kernel(u):
    s = S

    def body(u_ref, o_ref, zlo_ref, zhi_ref, rzl_ref, rzh_ref,
             bxl, bxh, byl, byh, send_sems, recv_sems):
        mx = lax.axis_index("x")
        my = lax.axis_index("y")
        mz = lax.axis_index("z")

        has_xm = mx > 0
        has_xp = mx < NX - 1
        has_ym = my > 0
        has_yp = my < NY - 1
        has_zm = mz > 0
        has_zp = mz < NZ - 1

        barrier_sem = pltpu.get_barrier_semaphore()

        def sig(dev):
            pl.semaphore_signal(
                barrier_sem, inc=1, device_id=dev,
                device_id_type=pl.DeviceIdType.MESH,
            )

        @pl.when(has_xm)
        def _():
            sig((mx - 1, my, mz))

        @pl.when(has_xp)
        def _():
            sig((mx + 1, my, mz))

        @pl.when(has_ym)
        def _():
            sig((mx, my - 1, mz))

        @pl.when(has_yp)
        def _():
            sig((mx, my + 1, mz))

        @pl.when(has_zm)
        def _():
            sig((mx, my, mz - 1))

        @pl.when(has_zp)
        def _():
            sig((mx, my, mz + 1))

        zlo_ref[...] = u_ref[:, :, 0]
        zhi_ref[...] = u_ref[:, :, s - 1]

        pl.semaphore_wait(barrier_sem, 3)

        @pl.when(has_zm & has_zp)
        def _():
            pl.semaphore_wait(barrier_sem, 1)

        def mk(src, dst, ssem_i, rsem_i, dev):
            return pltpu.make_async_remote_copy(
                src_ref=src,
                dst_ref=dst,
                send_sem=send_sems.at[ssem_i],
                recv_sem=recv_sems.at[rsem_i],
                device_id=dev,
                device_id_type=pl.DeviceIdType.MESH,
            )

        @pl.when(has_xm)
        def _():
            mk(u_ref.at[pl.ds(0, 1), :, :], bxh.at[pl.ds(0, 1), :, :],
               0, 1, (mx - 1, my, mz)).start()

        @pl.when(has_xp)
        def _():
            mk(u_ref.at[pl.ds(s - 1, 1), :, :], bxl.at[pl.ds(s - 1, 1), :, :],
               1, 0, (mx + 1, my, mz)).start()

        @pl.when(has_ym)
        def _():
            mk(u_ref.at[:, pl.ds(0, 1), :], byh.at[:, pl.ds(0, 1), :],
               2, 3, (mx, my - 1, mz)).start()

        @pl.when(has_yp)
        def _():
            mk(u_ref.at[:, pl.ds(s - 1, 1), :], byl.at[:, pl.ds(s - 1, 1), :],
               3, 2, (mx, my + 1, mz)).start()

        @pl.when(has_zm)
        def _():
            mk(zlo_ref, rzh_ref, 4, 5, (mx, my, mz - 1)).start()

        @pl.when(has_zp)
        def _():
            mk(zhi_ref, rzl_ref, 5, 4, (mx, my, mz + 1)).start()

        o_ref[...] = -6.0 * u_ref[...]
        o_ref[pl.ds(1, s - 1), :, :] = (
            o_ref[pl.ds(1, s - 1), :, :] + u_ref[pl.ds(0, s - 1), :, :]
        )
        o_ref[pl.ds(0, s - 1), :, :] = (
            o_ref[pl.ds(0, s - 1), :, :] + u_ref[pl.ds(1, s - 1), :, :]
        )
        o_ref[:, pl.ds(1, s - 1), :] = (
            o_ref[:, pl.ds(1, s - 1), :] + u_ref[:, pl.ds(0, s - 1), :]
        )
        o_ref[:, pl.ds(0, s - 1), :] = (
            o_ref[:, pl.ds(0, s - 1), :] + u_ref[:, pl.ds(1, s - 1), :]
        )
        o_ref[:, :, pl.ds(1, s - 1)] = (
            o_ref[:, :, pl.ds(1, s - 1)] + u_ref[:, :, pl.ds(0, s - 1)]
        )
        o_ref[:, :, pl.ds(0, s - 1)] = (
            o_ref[:, :, pl.ds(0, s - 1)] + u_ref[:, :, pl.ds(1, s - 1)]
        )

        def wait_recv(dst, rsem_i):
            pltpu.make_async_remote_copy(
                src_ref=dst,
                dst_ref=dst,
                send_sem=send_sems.at[rsem_i],
                recv_sem=recv_sems.at[rsem_i],
                device_id=(mx, my, mz),
                device_id_type=pl.DeviceIdType.MESH,
            ).wait_recv()

        @pl.when(has_xm)
        def _():
            wait_recv(bxl.at[pl.ds(s - 1, 1), :, :], 0)
            o_ref[pl.ds(0, 1), :, :] = (
                o_ref[pl.ds(0, 1), :, :] + bxl[pl.ds(s - 1, 1), :, :]
            )

        @pl.when(has_xp)
        def _():
            wait_recv(bxh.at[pl.ds(0, 1), :, :], 1)
            o_ref[pl.ds(s - 1, 1), :, :] = (
                o_ref[pl.ds(s - 1, 1), :, :] + bxh[pl.ds(0, 1), :, :]
            )

        @pl.when(has_ym)
        def _():
            wait_recv(byl.at[:, pl.ds(s - 1, 1), :], 2)
            o_ref[:, pl.ds(0, 1), :] = (
                o_ref[:, pl.ds(0, 1), :] + byl[:, pl.ds(s - 1, 1), :]
            )

        @pl.when(has_yp)
        def _():
            wait_recv(byh.at[:, pl.ds(0, 1), :], 3)
            o_ref[:, pl.ds(s - 1, 1), :] = (
                o_ref[:, pl.ds(s - 1, 1), :] + byh[:, pl.ds(0, 1), :]
            )

        @pl.when(has_zm)
        def _():
            wait_recv(rzl_ref, 4)
            o_ref[:, :, pl.ds(0, 1)] = (
                o_ref[:, :, pl.ds(0, 1)] + rzl_ref[...][:, :, None]
            )

        @pl.when(has_zp)
        def _():
            wait_recv(rzh_ref, 5)
            o_ref[:, :, pl.ds(s - 1, 1)] = (
                o_ref[:, :, pl.ds(s - 1, 1)] + rzh_ref[...][:, :, None]
            )

        @pl.when(mx == 0)
        def _():
            o_ref[pl.ds(0, 1), :, :] = jnp.zeros((1, s, s), jnp.float32)

        @pl.when(mx == NX - 1)
        def _():
            o_ref[pl.ds(s - 1, 1), :, :] = jnp.zeros((1, s, s), jnp.float32)

        @pl.when(my == 0)
        def _():
            o_ref[:, pl.ds(0, 1), :] = jnp.zeros((s, 1, s), jnp.float32)

        @pl.when(my == NY - 1)
        def _():
            o_ref[:, pl.ds(s - 1, 1), :] = jnp.zeros((s, 1, s), jnp.float32)

        @pl.when(mz == 0)
        def _():
            o_ref[:, :, pl.ds(0, 1)] = jnp.zeros((s, s, 1), jnp.float32)

        @pl.when(mz == NZ - 1)
        def _():
            o_ref[:, :, pl.ds(s - 1, 1)] = jnp.zeros((s, s, 1), jnp.float32)

        def wait_send(src, ssem_i):
            pltpu.make_async_remote_copy(
                src_ref=src,
                dst_ref=src,
                send_sem=send_sems.at[ssem_i],
                recv_sem=recv_sems.at[ssem_i],
                device_id=(mx, my, mz),
                device_id_type=pl.DeviceIdType.MESH,
            ).wait_send()

        @pl.when(has_xm)
        def _():
            wait_send(u_ref.at[pl.ds(0, 1), :, :], 0)

        @pl.when(has_xp)
        def _():
            wait_send(u_ref.at[pl.ds(s - 1, 1), :, :], 1)

        @pl.when(has_ym)
        def _():
            wait_send(u_ref.at[:, pl.ds(0, 1), :], 2)

        @pl.when(has_yp)
        def _():
            wait_send(u_ref.at[:, pl.ds(s - 1, 1), :], 3)

        @pl.when(has_zm)
        def _():
            wait_send(zlo_ref, 4)

        @pl.when(has_zp)
        def _():
            wait_send(zhi_ref, 5)

    return pl.pallas_call(
        body,
        out_shape=jax.ShapeDtypeStruct((s, s, s), jnp.float32),
        in_specs=[pl.BlockSpec(memory_space=pltpu.VMEM)],
        out_specs=pl.BlockSpec(memory_space=pltpu.VMEM),
        scratch_shapes=[
            pltpu.VMEM((s, s), jnp.float32),
            pltpu.VMEM((s, s), jnp.float32),
            pltpu.VMEM((s, s), jnp.float32),
            pltpu.VMEM((s, s), jnp.float32),
            pltpu.VMEM((s, s, s), jnp.float32),
            pltpu.VMEM((s, s, s), jnp.float32),
            pltpu.VMEM((s, s, s), jnp.float32),
            pltpu.VMEM((s, s, s), jnp.float32),
            pltpu.SemaphoreType.DMA((6,)),
            pltpu.SemaphoreType.DMA((6,)),
        ],
        compiler_params=pltpu.CompilerParams(collective_id=0),
    )(u)
